# Initial kernel scaffold; baseline (speedup 1.0000x reference)
import jax
import jax.numpy as jnp
from jax.experimental import pallas as pl


def kernel(class_logits, box_regression, proposals):
    raise NotImplementedError("write your pallas kernel here")


# speedup vs baseline: 21.7490x; 21.7490x over previous
import jax
import jax.numpy as jnp
import math
from jax.experimental import pallas as pl
from jax.experimental.pallas import tpu as pltpu

_B = 2
_N = 1000
_C = 91
_IMG = 800.0
_SCORE_THRESH = 0.05
_NMS_THRESH = 0.5
_DETS = 100
_MIN_SIZE = 0.01
_CLIP = math.log(1000.0 / 16.0)
_LANES = 128


def _fused(lg_ref, d_ref, props_ref, out_ref,
           s_ref, xo1_ref, yo1_ref, xo2_ref, yo2_ref, ar_ref,
           ux1_ref, uy1_ref, ux2_ref, uy2_ref):
    f32 = jnp.float32
    l = lg_ref[...]
    m = jnp.max(l, axis=1, keepdims=True)
    e = jnp.exp(l - m)
    p = e / jnp.sum(e, axis=1, keepdims=True)

    pr = props_ref[0]
    w = pr[:, 2:3] - pr[:, 0:1]
    h = pr[:, 3:4] - pr[:, 1:2]
    cx = pr[:, 0:1] + 0.5 * w
    cy = pr[:, 1:2] + 0.5 * h
    dx = d_ref[0] / 10.0
    dy = d_ref[1] / 10.0
    dw = jnp.minimum(d_ref[2] / 5.0, _CLIP)
    dh = jnp.minimum(d_ref[3] / 5.0, _CLIP)
    pcx = dx * w + cx
    pcy = dy * h + cy
    pw = jnp.exp(dw) * w
    ph = jnp.exp(dh) * h
    x1 = jnp.clip(pcx - 0.5 * pw, 0.0, _IMG)
    y1 = jnp.clip(pcy - 0.5 * ph, 0.0, _IMG)
    x2 = jnp.clip(pcx + 0.5 * pw, 0.0, _IMG)
    y2 = jnp.clip(pcy + 0.5 * ph, 0.0, _IMG)

    col = jax.lax.broadcasted_iota(jnp.int32, (_N, _LANES), 1)
    row = jax.lax.broadcasted_iota(jnp.int32, (_N, _LANES), 0)
    lin = row * _LANES + col
    colf = col.astype(f32)
    col_ok = (col >= 1) & (col <= _C - 1)
    mask = (p > _SCORE_THRESH) & ((x2 - x1) >= _MIN_SIZE) & \
           ((y2 - y1) >= _MIN_SIZE) & col_ok
    s_ref[...] = jnp.where(mask, p, -1.0)

    offs = colf * (_IMG + 2.0)
    xo1 = x1 + offs
    yo1 = y1 + offs
    xo2 = x2 + offs
    yo2 = y2 + offs
    xo1_ref[...] = xo1
    yo1_ref[...] = yo1
    xo2_ref[...] = xo2
    yo2_ref[...] = yo2
    ar_ref[...] = (xo2 - xo1) * (yo2 - yo1)
    ux1_ref[...] = x1
    uy1_ref[...] = y1
    ux2_ref[...] = x2
    uy2_ref[...] = y2

    lane = jax.lax.broadcasted_iota(jnp.int32, (1, _LANES), 1)

    def body(t, carry):
        s = s_ref[...]
        mx = jnp.max(s)
        idx = jnp.min(jnp.where(s == mx, lin, jnp.int32(1 << 30)))
        rr = idx // _LANES
        cc = idx % _LANES
        ok = mx > 0.0

        def pick(ref):
            rowv = ref[pl.ds(rr, 1), :]
            return jnp.sum(jnp.where(lane == cc, rowv, 0.0))

        bx1 = pick(xo1_ref)
        by1 = pick(yo1_ref)
        bx2 = pick(xo2_ref)
        by2 = pick(yo2_ref)
        ref_area = (bx2 - bx1) * (by2 - by1)

        xx1 = jnp.maximum(bx1, xo1_ref[...])
        yy1 = jnp.maximum(by1, yo1_ref[...])
        xx2 = jnp.minimum(bx2, xo2_ref[...])
        yy2 = jnp.minimum(by2, yo2_ref[...])
        inter = jnp.maximum(xx2 - xx1, 0.0) * jnp.maximum(yy2 - yy1, 0.0)
        union = ar_ref[...] + ref_area - inter
        iou = jnp.where(union > 0.0, inter / jnp.maximum(union, 1e-9), 0.0)
        supp = (iou > _NMS_THRESH) & ok
        s_ref[...] = jnp.where(supp | (lin == idx), -1.0, s)

        vf = jnp.where(ok, 1.0, 0.0).astype(f32)
        vals = [pick(ux1_ref) * vf, pick(uy1_ref) * vf,
                pick(ux2_ref) * vf, pick(uy2_ref) * vf,
                jnp.where(ok, mx, 0.0),
                jnp.where(ok, cc, 0).astype(f32),
                jnp.where(ok, rr * 90 + cc - 1, 0).astype(f32),
                vf]
        rowout = jnp.zeros((1, _LANES), f32)
        for j, v in enumerate(vals):
            rowout = jnp.where(lane == j, v, rowout)
        out_ref[0, pl.ds(t, 1), :] = rowout
        return carry

    jax.lax.fori_loop(0, _DETS, body, 0)


def kernel(class_logits, box_regression, proposals):
    lg = jnp.pad(class_logits.astype(jnp.float32), ((0, 0), (0, _LANES - _C)),
                 constant_values=-1e9)
    rel = box_regression.reshape(_B * _N, _C, 4)
    d = jnp.transpose(rel, (2, 0, 1))
    d = jnp.pad(d, ((0, 0), (0, 0), (0, _LANES - _C)))

    out = pl.pallas_call(
        _fused,
        grid=(_B,),
        in_specs=[
            pl.BlockSpec((_N, _LANES), lambda i: (i, 0)),
            pl.BlockSpec((4, _N, _LANES), lambda i: (0, i, 0)),
            pl.BlockSpec((1, _N, 4), lambda i: (i, 0, 0)),
        ],
        out_specs=pl.BlockSpec((1, _LANES, _LANES), lambda i: (i, 0, 0)),
        out_shape=jax.ShapeDtypeStruct((_B, _LANES, _LANES), jnp.float32),
        scratch_shapes=[pltpu.VMEM((_N, _LANES), jnp.float32)] * 10,
    )(lg, d, proposals)

    res = out[:, :_DETS, :]
    sel_boxes = res[..., 0:4]
    sel_scores = res[..., 4]
    sel_labels = res[..., 5].astype(jnp.int32)
    keep = res[..., 6].astype(jnp.int32)
    valid = res[..., 7] > 0.5
    return sel_boxes, sel_scores, sel_labels, keep, valid


# speedup vs baseline: 26.3074x; 1.2096x over previous
import jax
import jax.numpy as jnp
import math
from jax.experimental import pallas as pl
from jax.experimental.pallas import tpu as pltpu

_B = 2
_N = 1000
_NP = 1024
_C = 91
_CP = 128
_IMG = 800.0
_SCORE_THRESH = 0.05
_NMS_THRESH = 0.5
_DETS = 100
_MIN_SIZE = 0.01
_CLIP = math.log(1000.0 / 16.0)


def _fused(lg_ref, d_ref, props_ref, out_ref,
           s_ref, xo1_ref, yo1_ref, xo2_ref, yo2_ref, ar_ref,
           ux1_ref, uy1_ref, ux2_ref, uy2_ref):
    f32 = jnp.float32
    l = lg_ref[0]
    m = jnp.max(l, axis=0, keepdims=True)
    e = jnp.exp(l - m)
    p = e / jnp.sum(e, axis=0, keepdims=True)

    pr = props_ref[0]
    p0 = pr[0:1, :]
    p1 = pr[1:2, :]
    p2 = pr[2:3, :]
    p3 = pr[3:4, :]
    w = p2 - p0
    h = p3 - p1
    cx = p0 + 0.5 * w
    cy = p1 + 0.5 * h
    dx = d_ref[0, 0] / 10.0
    dy = d_ref[0, 1] / 10.0
    dw = jnp.minimum(d_ref[0, 2] / 5.0, _CLIP)
    dh = jnp.minimum(d_ref[0, 3] / 5.0, _CLIP)
    pcx = dx * w + cx
    pcy = dy * h + cy
    pw = jnp.exp(dw) * w
    ph = jnp.exp(dh) * h
    x1 = jnp.clip(pcx - 0.5 * pw, 0.0, _IMG)
    y1 = jnp.clip(pcy - 0.5 * ph, 0.0, _IMG)
    x2 = jnp.clip(pcx + 0.5 * pw, 0.0, _IMG)
    y2 = jnp.clip(pcy + 0.5 * ph, 0.0, _IMG)

    crow = jax.lax.broadcasted_iota(jnp.int32, (_CP, _NP), 0)
    ncol = jax.lax.broadcasted_iota(jnp.int32, (_CP, _NP), 1)
    ok_pos = (crow >= 1) & (crow <= _C - 1) & (ncol < _N)
    mask = (p > _SCORE_THRESH) & ((x2 - x1) >= _MIN_SIZE) & \
           ((y2 - y1) >= _MIN_SIZE) & ok_pos
    s_ref[...] = jnp.where(mask, p, -1.0)

    offs = crow.astype(f32) * (_IMG + 2.0)
    xo1 = x1 + offs
    yo1 = y1 + offs
    xo2 = x2 + offs
    yo2 = y2 + offs
    xo1_ref[...] = xo1
    yo1_ref[...] = yo1
    xo2_ref[...] = xo2
    yo2_ref[...] = yo2
    ar_ref[...] = (xo2 - xo1) * (yo2 - yo1)
    ux1_ref[...] = x1
    uy1_ref[...] = y1
    ux2_ref[...] = x2
    uy2_ref[...] = y2

    lane = jax.lax.broadcasted_iota(jnp.int32, (1, _CP), 1)
    nlane = jax.lax.broadcasted_iota(jnp.int32, (1, _NP), 1)

    def init_m(c, M):
        rmax = jnp.max(s_ref[pl.ds(c, 1), :])
        return jnp.where(lane == c, rmax, M)

    M0 = jax.lax.fori_loop(0, _CP, init_m, jnp.full((1, _CP), -1.0, f32))

    def body(t, M):
        mx = jnp.max(M)
        cc = jnp.min(jnp.where(M == mx, lane, jnp.int32(1 << 30)))
        srow = s_ref[pl.ds(cc, 1), :]
        nn = jnp.min(jnp.where(srow == mx, nlane, jnp.int32(1 << 30)))
        ok = mx > 0.0
        sel = nlane == nn

        def pick(ref):
            return jnp.sum(jnp.where(sel, ref[pl.ds(cc, 1), :], 0.0))

        ro1 = xo1_ref[pl.ds(cc, 1), :]
        ro2 = yo1_ref[pl.ds(cc, 1), :]
        ro3 = xo2_ref[pl.ds(cc, 1), :]
        ro4 = yo2_ref[pl.ds(cc, 1), :]
        bx1 = jnp.sum(jnp.where(sel, ro1, 0.0))
        by1 = jnp.sum(jnp.where(sel, ro2, 0.0))
        bx2 = jnp.sum(jnp.where(sel, ro3, 0.0))
        by2 = jnp.sum(jnp.where(sel, ro4, 0.0))
        ref_area = (bx2 - bx1) * (by2 - by1)

        xx1 = jnp.maximum(bx1, ro1)
        yy1 = jnp.maximum(by1, ro2)
        xx2 = jnp.minimum(bx2, ro3)
        yy2 = jnp.minimum(by2, ro4)
        inter = jnp.maximum(xx2 - xx1, 0.0) * jnp.maximum(yy2 - yy1, 0.0)
        union = ar_ref[pl.ds(cc, 1), :] + ref_area - inter
        iou = jnp.where(union > 0.0, inter / jnp.maximum(union, 1e-9), 0.0)
        supp = (iou > _NMS_THRESH) & ok
        srow_new = jnp.where(supp | sel, -1.0, srow)
        s_ref[pl.ds(cc, 1), :] = srow_new
        M_new = jnp.where(lane == cc, jnp.max(srow_new), M)

        vf = jnp.where(ok, 1.0, 0.0).astype(f32)
        vals = [pick(ux1_ref) * vf, pick(uy1_ref) * vf,
                pick(ux2_ref) * vf, pick(uy2_ref) * vf,
                jnp.where(ok, mx, 0.0),
                jnp.where(ok, cc, 0).astype(f32),
                jnp.where(ok, nn * 90 + cc - 1, 0).astype(f32),
                vf]
        rowout = jnp.zeros((1, _CP), f32)
        for j, v in enumerate(vals):
            rowout = jnp.where(lane == j, v, rowout)
        out_ref[0, pl.ds(t, 1), :] = rowout
        return M_new

    jax.lax.fori_loop(0, _DETS, body, M0)


def kernel(class_logits, box_regression, proposals):
    f32 = jnp.float32
    lg = class_logits.astype(f32).reshape(_B, _N, _C).transpose(0, 2, 1)
    lg = jnp.pad(lg, ((0, 0), (0, _CP - _C), (0, _NP - _N)),
                 constant_values=-1e9)
    d = box_regression.astype(f32).reshape(_B, _N, _C, 4).transpose(0, 3, 2, 1)
    d = jnp.pad(d, ((0, 0), (0, 0), (0, _CP - _C), (0, _NP - _N)))
    pr = proposals.astype(f32).transpose(0, 2, 1)
    pr = jnp.pad(pr, ((0, 0), (0, 4), (0, _NP - _N)))

    out = pl.pallas_call(
        _fused,
        grid=(_B,),
        in_specs=[
            pl.BlockSpec((1, _CP, _NP), lambda i: (i, 0, 0)),
            pl.BlockSpec((1, 4, _CP, _NP), lambda i: (i, 0, 0, 0)),
            pl.BlockSpec((1, 8, _NP), lambda i: (i, 0, 0)),
        ],
        out_specs=pl.BlockSpec((1, _CP, _CP), lambda i: (i, 0, 0)),
        out_shape=jax.ShapeDtypeStruct((_B, _CP, _CP), f32),
        scratch_shapes=[pltpu.VMEM((_CP, _NP), f32)] * 10,
    )(lg, d, pr)

    res = out[:, :_DETS, :]
    sel_boxes = res[..., 0:4]
    sel_scores = res[..., 4]
    sel_labels = res[..., 5].astype(jnp.int32)
    keep = res[..., 6].astype(jnp.int32)
    valid = res[..., 7] > 0.5
    return sel_boxes, sel_scores, sel_labels, keep, valid


# speedup vs baseline: 30.3937x; 1.1553x over previous
import jax
import jax.numpy as jnp
import math
from jax.experimental import pallas as pl
from jax.experimental.pallas import tpu as pltpu

_B = 2
_N = 1000
_NP = 1024
_C = 91
_CP = 128
_R = _B * _CP
_IMG = 800.0
_SCORE_THRESH = 0.05
_NMS_THRESH = 0.5
_DETS = 100
_MIN_SIZE = 0.01
_CLIP = math.log(1000.0 / 16.0)


def _fused(lg_ref, d_ref, props_ref, out_ref,
           s_ref, xo1_ref, yo1_ref, xo2_ref, yo2_ref, ar_ref,
           ux1_ref, uy1_ref, ux2_ref, uy2_ref):
    f32 = jnp.float32
    l = lg_ref[...]
    l0 = l.reshape(_B, _CP, _NP)
    m = jnp.max(l0, axis=1, keepdims=True)
    e = jnp.exp(l0 - m)
    p = (e / jnp.sum(e, axis=1, keepdims=True)).reshape(_R, _NP)

    pr = props_ref[...]
    pr0 = pr.reshape(_B, 8, _NP)
    p0 = pr0[:, 0:1, :]
    p1 = pr0[:, 1:2, :]
    p2 = pr0[:, 2:3, :]
    p3 = pr0[:, 3:4, :]
    w = (p2 - p0)
    h = (p3 - p1)
    cx = p0 + 0.5 * w
    cy = p1 + 0.5 * h
    d = d_ref[...].reshape(4, _B, _CP, _NP)
    dx = d[0] / 10.0
    dy = d[1] / 10.0
    dw = jnp.minimum(d[2] / 5.0, _CLIP)
    dh = jnp.minimum(d[3] / 5.0, _CLIP)
    pcx = dx * w + cx
    pcy = dy * h + cy
    pw = jnp.exp(dw) * w
    ph = jnp.exp(dh) * h
    x1 = jnp.clip(pcx - 0.5 * pw, 0.0, _IMG).reshape(_R, _NP)
    y1 = jnp.clip(pcy - 0.5 * ph, 0.0, _IMG).reshape(_R, _NP)
    x2 = jnp.clip(pcx + 0.5 * pw, 0.0, _IMG).reshape(_R, _NP)
    y2 = jnp.clip(pcy + 0.5 * ph, 0.0, _IMG).reshape(_R, _NP)

    rrow = jax.lax.broadcasted_iota(jnp.int32, (_R, _NP), 0)
    crow = jax.lax.rem(rrow, _CP)
    ncol = jax.lax.broadcasted_iota(jnp.int32, (_R, _NP), 1)
    ok_pos = (crow >= 1) & (crow <= _C - 1) & (ncol < _N)
    mask = (p > _SCORE_THRESH) & ((x2 - x1) >= _MIN_SIZE) & \
           ((y2 - y1) >= _MIN_SIZE) & ok_pos
    s = jnp.where(mask, p, -1.0)
    s_ref[...] = s

    offs = crow.astype(f32) * (_IMG + 2.0)
    xo1 = x1 + offs
    yo1 = y1 + offs
    xo2 = x2 + offs
    yo2 = y2 + offs
    xo1_ref[...] = xo1
    yo1_ref[...] = yo1
    xo2_ref[...] = xo2
    yo2_ref[...] = yo2
    ar_ref[...] = (xo2 - xo1) * (yo2 - yo1)
    ux1_ref[...] = x1
    uy1_ref[...] = y1
    ux2_ref[...] = x2
    uy2_ref[...] = y2

    lane = jax.lax.broadcasted_iota(jnp.int32, (1, _R), 1)
    nlane = jax.lax.broadcasted_iota(jnp.int32, (1, _NP), 1)

    M0 = jnp.max(s, axis=1).reshape(1, _R)

    def one_image(b, M, t):
        Mb = jnp.where((lane >= b * _CP) & (lane < (b + 1) * _CP), M, -2.0)
        mx = jnp.max(Mb)
        cc = jnp.min(jnp.where(Mb == mx, lane, jnp.int32(1 << 30)))
        srow = s_ref[pl.ds(cc, 1), :]
        nn = jnp.min(jnp.where(srow == mx, nlane, jnp.int32(1 << 30)))
        ok = mx > 0.0
        sel = nlane == nn

        ro1 = xo1_ref[pl.ds(cc, 1), :]
        ro2 = yo1_ref[pl.ds(cc, 1), :]
        ro3 = xo2_ref[pl.ds(cc, 1), :]
        ro4 = yo2_ref[pl.ds(cc, 1), :]
        bx1 = jnp.sum(jnp.where(sel, ro1, 0.0))
        by1 = jnp.sum(jnp.where(sel, ro2, 0.0))
        bx2 = jnp.sum(jnp.where(sel, ro3, 0.0))
        by2 = jnp.sum(jnp.where(sel, ro4, 0.0))
        ref_area = (bx2 - bx1) * (by2 - by1)

        xx1 = jnp.maximum(bx1, ro1)
        yy1 = jnp.maximum(by1, ro2)
        xx2 = jnp.minimum(bx2, ro3)
        yy2 = jnp.minimum(by2, ro4)
        inter = jnp.maximum(xx2 - xx1, 0.0) * jnp.maximum(yy2 - yy1, 0.0)
        union = ar_ref[pl.ds(cc, 1), :] + ref_area - inter
        iou = jnp.where(union > 0.0, inter / jnp.maximum(union, 1e-9), 0.0)
        supp = (iou > _NMS_THRESH) & ok
        srow_new = jnp.where(supp | sel, -1.0, srow)
        s_ref[pl.ds(cc, 1), :] = srow_new
        M_new = jnp.where(lane == cc, jnp.max(srow_new), M)

        def pick(ref):
            return jnp.sum(jnp.where(sel, ref[pl.ds(cc, 1), :], 0.0))

        vf = jnp.where(ok, 1.0, 0.0).astype(f32)
        cls = cc - b * _CP
        vals = [pick(ux1_ref) * vf, pick(uy1_ref) * vf,
                pick(ux2_ref) * vf, pick(uy2_ref) * vf,
                jnp.where(ok, mx, 0.0),
                jnp.where(ok, cls, 0).astype(f32),
                jnp.where(ok, nn * 90 + cls - 1, 0).astype(f32),
                vf]
        rowout = jnp.zeros((1, _R), f32)
        for j, v in enumerate(vals):
            rowout = jnp.where(lane == j, v, rowout)
        out_ref[b, pl.ds(t, 1), :] = rowout[:, :_CP]
        return M_new

    def body(t, M):
        M = one_image(0, M, t)
        M = one_image(1, M, t)
        return M

    jax.lax.fori_loop(0, _DETS, body, M0)


def kernel(class_logits, box_regression, proposals):
    f32 = jnp.float32
    lg = class_logits.astype(f32).reshape(_B, _N, _C).transpose(0, 2, 1)
    lg = jnp.pad(lg, ((0, 0), (0, _CP - _C), (0, _NP - _N)),
                 constant_values=-1e9).reshape(_R, _NP)
    d = box_regression.astype(f32).reshape(_B, _N, _C, 4).transpose(3, 0, 2, 1)
    d = jnp.pad(d, ((0, 0), (0, 0), (0, _CP - _C), (0, _NP - _N)))
    d = d.reshape(4 * _R, _NP)
    pr = proposals.astype(f32).transpose(0, 2, 1)
    pr = jnp.pad(pr, ((0, 0), (0, 4), (0, _NP - _N))).reshape(_B * 8, _NP)

    out = pl.pallas_call(
        _fused,
        out_shape=jax.ShapeDtypeStruct((_B, _CP, _CP), f32),
        scratch_shapes=[pltpu.VMEM((_R, _NP), f32)] * 10,
    )(lg, d, pr)

    res = out[:, :_DETS, :]
    sel_boxes = res[..., 0:4]
    sel_scores = res[..., 4]
    sel_labels = res[..., 5].astype(jnp.int32)
    keep = res[..., 6].astype(jnp.int32)
    valid = res[..., 7] > 0.5
    return sel_boxes, sel_scores, sel_labels, keep, valid


# speedup vs baseline: 32.4622x; 1.0681x over previous
import jax
import jax.numpy as jnp
import math
from jax.experimental import pallas as pl
from jax.experimental.pallas import tpu as pltpu

_B = 2
_N = 1000
_NP = 1024
_C = 91
_CP = 128
_R = _B * _CP
_IMG = 800.0
_SCORE_THRESH = 0.05
_NMS_THRESH = 0.5
_DETS = 100
_MIN_SIZE = 0.01
_CLIP = math.log(1000.0 / 16.0)


def _fused(lg_ref, d_ref, props_ref, out_ref, *scratch):
    f32 = jnp.float32
    refs = [scratch[b * 10:(b + 1) * 10] for b in range(_B)]
    l = lg_ref[...].reshape(_B, _CP, _NP)
    m = jnp.max(l, axis=1, keepdims=True)
    e = jnp.exp(l - m)
    p = e / jnp.sum(e, axis=1, keepdims=True)

    pr = props_ref[...].reshape(_B, 8, _NP)
    p0 = pr[:, 0:1, :]
    p1 = pr[:, 1:2, :]
    p2 = pr[:, 2:3, :]
    p3 = pr[:, 3:4, :]
    w = (p2 - p0)
    h = (p3 - p1)
    cx = p0 + 0.5 * w
    cy = p1 + 0.5 * h
    d = d_ref[...].reshape(4, _B, _CP, _NP)
    dx = d[0] / 10.0
    dy = d[1] / 10.0
    dw = jnp.minimum(d[2] / 5.0, _CLIP)
    dh = jnp.minimum(d[3] / 5.0, _CLIP)
    pcx = dx * w + cx
    pcy = dy * h + cy
    pw = jnp.exp(dw) * w
    ph = jnp.exp(dh) * h
    x1 = jnp.clip(pcx - 0.5 * pw, 0.0, _IMG)
    y1 = jnp.clip(pcy - 0.5 * ph, 0.0, _IMG)
    x2 = jnp.clip(pcx + 0.5 * pw, 0.0, _IMG)
    y2 = jnp.clip(pcy + 0.5 * ph, 0.0, _IMG)

    crow = jax.lax.broadcasted_iota(jnp.int32, (_CP, _NP), 0)
    ncol = jax.lax.broadcasted_iota(jnp.int32, (_CP, _NP), 1)
    ok_pos = (crow >= 1) & (crow <= _C - 1) & (ncol < _N)
    offs = crow.astype(f32) * (_IMG + 2.0)

    M0s = []
    for b in range(_B):
        (s_ref, xo1_ref, yo1_ref, xo2_ref, yo2_ref, ar_ref,
         ux1_ref, uy1_ref, ux2_ref, uy2_ref) = refs[b]
        mask = (p[b] > _SCORE_THRESH) & ((x2[b] - x1[b]) >= _MIN_SIZE) & \
               ((y2[b] - y1[b]) >= _MIN_SIZE) & ok_pos
        s = jnp.where(mask, p[b], -1.0)
        s_ref[...] = s
        xo1 = x1[b] + offs
        yo1 = y1[b] + offs
        xo2 = x2[b] + offs
        yo2 = y2[b] + offs
        xo1_ref[...] = xo1
        yo1_ref[...] = yo1
        xo2_ref[...] = xo2
        yo2_ref[...] = yo2
        ar_ref[...] = (xo2 - xo1) * (yo2 - yo1)
        ux1_ref[...] = x1[b]
        uy1_ref[...] = y1[b]
        ux2_ref[...] = x2[b]
        uy2_ref[...] = y2[b]
        M0s.append(jnp.max(s, axis=1).reshape(1, _CP))

    lane = jax.lax.broadcasted_iota(jnp.int32, (1, _CP), 1)
    nlane = jax.lax.broadcasted_iota(jnp.int32, (1, _NP), 1)

    def one_image(b, M, t):
        (s_ref, xo1_ref, yo1_ref, xo2_ref, yo2_ref, ar_ref,
         ux1_ref, uy1_ref, ux2_ref, uy2_ref) = refs[b]
        mx = jnp.max(M)
        cc = jnp.min(jnp.where(M == mx, lane, jnp.int32(1 << 30)))
        srow = s_ref[pl.ds(cc, 1), :]
        nn = jnp.min(jnp.where(srow == mx, nlane, jnp.int32(1 << 30)))
        ok = mx > 0.0
        sel = nlane == nn

        ro1 = xo1_ref[pl.ds(cc, 1), :]
        ro2 = yo1_ref[pl.ds(cc, 1), :]
        ro3 = xo2_ref[pl.ds(cc, 1), :]
        ro4 = yo2_ref[pl.ds(cc, 1), :]
        bx1 = jnp.sum(jnp.where(sel, ro1, 0.0))
        by1 = jnp.sum(jnp.where(sel, ro2, 0.0))
        bx2 = jnp.sum(jnp.where(sel, ro3, 0.0))
        by2 = jnp.sum(jnp.where(sel, ro4, 0.0))
        ref_area = (bx2 - bx1) * (by2 - by1)

        xx1 = jnp.maximum(bx1, ro1)
        yy1 = jnp.maximum(by1, ro2)
        xx2 = jnp.minimum(bx2, ro3)
        yy2 = jnp.minimum(by2, ro4)
        inter = jnp.maximum(xx2 - xx1, 0.0) * jnp.maximum(yy2 - yy1, 0.0)
        union = ar_ref[pl.ds(cc, 1), :] + ref_area - inter
        iou = jnp.where(union > 0.0, inter / jnp.maximum(union, 1e-9), 0.0)
        supp = (iou > _NMS_THRESH) & ok
        srow_new = jnp.where(supp | sel, -1.0, srow)
        s_ref[pl.ds(cc, 1), :] = srow_new
        M_new = jnp.where(lane == cc, jnp.max(srow_new), M)

        def pick(ref):
            return jnp.sum(jnp.where(sel, ref[pl.ds(cc, 1), :], 0.0))

        vf = jnp.where(ok, 1.0, 0.0).astype(f32)
        vals = [pick(ux1_ref) * vf, pick(uy1_ref) * vf,
                pick(ux2_ref) * vf, pick(uy2_ref) * vf,
                jnp.where(ok, mx, 0.0),
                jnp.where(ok, cc, 0).astype(f32),
                jnp.where(ok, nn * 90 + cc - 1, 0).astype(f32),
                vf]
        rowout = jnp.zeros((1, _CP), f32)
        for j, v in enumerate(vals):
            rowout = jnp.where(lane == j, v, rowout)
        out_ref[b, pl.ds(t, 1), :] = rowout
        return M_new

    def body(t, Ms):
        return tuple(one_image(b, Ms[b], t) for b in range(_B))

    jax.lax.fori_loop(0, _DETS, body, tuple(M0s))


def kernel(class_logits, box_regression, proposals):
    f32 = jnp.float32
    lg = class_logits.astype(f32).reshape(_B, _N, _C).transpose(0, 2, 1)
    lg = jnp.pad(lg, ((0, 0), (0, _CP - _C), (0, _NP - _N)),
                 constant_values=-1e9).reshape(_R, _NP)
    d = box_regression.astype(f32).reshape(_B, _N, _C, 4).transpose(3, 0, 2, 1)
    d = jnp.pad(d, ((0, 0), (0, 0), (0, _CP - _C), (0, _NP - _N)))
    d = d.reshape(4 * _R, _NP)
    pr = proposals.astype(f32).transpose(0, 2, 1)
    pr = jnp.pad(pr, ((0, 0), (0, 4), (0, _NP - _N))).reshape(_B * 8, _NP)

    out = pl.pallas_call(
        _fused,
        out_shape=jax.ShapeDtypeStruct((_B, _CP, _CP), f32),
        scratch_shapes=[pltpu.VMEM((_CP, _NP), f32)] * (10 * _B),
    )(lg, d, pr)

    res = out[:, :_DETS, :]
    sel_boxes = res[..., 0:4]
    sel_scores = res[..., 4]
    sel_labels = res[..., 5].astype(jnp.int32)
    keep = res[..., 6].astype(jnp.int32)
    valid = res[..., 7] > 0.5
    return sel_boxes, sel_scores, sel_labels, keep, valid


# speedup vs baseline: 44.0385x; 1.3566x over previous
import jax
import jax.numpy as jnp
import math
from jax.experimental import pallas as pl
from jax.experimental.pallas import tpu as pltpu

_B = 2
_N = 1000
_NP = 1024
_C = 91
_CP = 128
_IMG = 800.0
_SCORE_THRESH = 0.05
_NMS_THRESH = 0.5
_DETS = 100
_MIN_SIZE = 0.01
_CLIP = math.log(1000.0 / 16.0)


def _fused(lg_ref, d_ref, props_ref, out_ref, *scratch):
    f32 = jnp.float32
    refs = [scratch[b * 6:(b + 1) * 6] for b in range(_B)]
    l = lg_ref[...].reshape(_B, _CP, 8, 128)
    m = jnp.max(l, axis=1, keepdims=True)
    e = jnp.exp(l - m)
    p = e / jnp.sum(e, axis=1, keepdims=True)

    pr = props_ref[...].reshape(_B, 4, 1, 8, 128)
    p0 = pr[:, 0]
    p1 = pr[:, 1]
    p2 = pr[:, 2]
    p3 = pr[:, 3]
    w = (p2 - p0)
    h = (p3 - p1)
    cx = p0 + 0.5 * w
    cy = p1 + 0.5 * h
    d = d_ref[...].reshape(4, _B, _CP, 8, 128)
    dx = d[0] / 10.0
    dy = d[1] / 10.0
    dw = jnp.minimum(d[2] / 5.0, _CLIP)
    dh = jnp.minimum(d[3] / 5.0, _CLIP)
    pcx = dx * w + cx
    pcy = dy * h + cy
    pw = jnp.exp(dw) * w
    ph = jnp.exp(dh) * h
    x1 = jnp.clip(pcx - 0.5 * pw, 0.0, _IMG)
    y1 = jnp.clip(pcy - 0.5 * ph, 0.0, _IMG)
    x2 = jnp.clip(pcx + 0.5 * pw, 0.0, _IMG)
    y2 = jnp.clip(pcy + 0.5 * ph, 0.0, _IMG)

    crow = jax.lax.broadcasted_iota(jnp.int32, (_CP, 8, 128), 0)
    nidx = jax.lax.broadcasted_iota(jnp.int32, (_CP, 8, 128), 1) * 128 + \
        jax.lax.broadcasted_iota(jnp.int32, (_CP, 8, 128), 2)
    ok_pos = (crow >= 1) & (crow <= _C - 1) & (nidx < _N)
    offs = crow.astype(f32) * (_IMG + 2.0)

    M0s = []
    for b in range(_B):
        s_ref, xo1_ref, yo1_ref, xo2_ref, yo2_ref, ar_ref = refs[b]
        mask = (p[b] > _SCORE_THRESH) & ((x2[b] - x1[b]) >= _MIN_SIZE) & \
               ((y2[b] - y1[b]) >= _MIN_SIZE) & ok_pos
        s = jnp.where(mask, p[b], -1.0)
        s_ref[...] = s.reshape(_CP * 8, 128)
        xo1 = x1[b] + offs
        yo1 = y1[b] + offs
        xo2 = x2[b] + offs
        yo2 = y2[b] + offs
        xo1_ref[...] = xo1.reshape(_CP * 8, 128)
        yo1_ref[...] = yo1.reshape(_CP * 8, 128)
        xo2_ref[...] = xo2.reshape(_CP * 8, 128)
        yo2_ref[...] = yo2.reshape(_CP * 8, 128)
        ar_ref[...] = ((xo2 - xo1) * (yo2 - yo1)).reshape(_CP * 8, 128)
        M0s.append(jnp.max(s, axis=(1, 2)).reshape(1, _CP))

    lane = jax.lax.broadcasted_iota(jnp.int32, (1, _CP), 1)
    tidx = jax.lax.broadcasted_iota(jnp.int32, (8, 128), 0) * 128 + \
        jax.lax.broadcasted_iota(jnp.int32, (8, 128), 1)

    def one_image(b, M, t):
        s_ref, xo1_ref, yo1_ref, xo2_ref, yo2_ref, ar_ref = refs[b]
        mx = jnp.max(M)
        cc = jnp.min(jnp.where(M == mx, lane, jnp.int32(1 << 30)))
        base = cc * 8
        srow = s_ref[pl.ds(base, 8), :]
        eq = srow == mx
        ok = mx > 0.0

        ro1 = xo1_ref[pl.ds(base, 8), :]
        ro2 = yo1_ref[pl.ds(base, 8), :]
        ro3 = xo2_ref[pl.ds(base, 8), :]
        ro4 = yo2_ref[pl.ds(base, 8), :]
        bx1 = jnp.sum(jnp.where(eq, ro1, 0.0))
        by1 = jnp.sum(jnp.where(eq, ro2, 0.0))
        bx2 = jnp.sum(jnp.where(eq, ro3, 0.0))
        by2 = jnp.sum(jnp.where(eq, ro4, 0.0))
        ref_area = (bx2 - bx1) * (by2 - by1)

        xx1 = jnp.maximum(bx1, ro1)
        yy1 = jnp.maximum(by1, ro2)
        xx2 = jnp.minimum(bx2, ro3)
        yy2 = jnp.minimum(by2, ro4)
        inter = jnp.maximum(xx2 - xx1, 0.0) * jnp.maximum(yy2 - yy1, 0.0)
        union = ar_ref[pl.ds(base, 8), :] + ref_area - inter
        iou = jnp.where(union > 0.0, inter / jnp.maximum(union, 1e-9), 0.0)
        supp = (iou > _NMS_THRESH) & ok
        srow_new = jnp.where(supp, -1.0, srow)
        s_ref[pl.ds(base, 8), :] = srow_new
        M_new = jnp.where(lane == cc, jnp.max(srow_new), M)

        nn = jnp.min(jnp.where(eq, tidx, jnp.int32(1 << 30)))
        off_c = cc.astype(f32) * (_IMG + 2.0)
        vf = jnp.where(ok, 1.0, 0.0).astype(f32)
        vals = [(bx1 - off_c) * vf, (by1 - off_c) * vf,
                (bx2 - off_c) * vf, (by2 - off_c) * vf,
                jnp.where(ok, mx, 0.0),
                jnp.where(ok, cc, 0).astype(f32),
                jnp.where(ok, nn * 90 + cc - 1, 0).astype(f32),
                vf]
        for j, v in enumerate(vals):
            out_ref[b, pl.ds(t, 1), j:j + 1] = v.reshape(1, 1)
        return M_new

    def body(t, Ms):
        return tuple(one_image(b, Ms[b], t) for b in range(_B))

    jax.lax.fori_loop(0, _DETS, body, tuple(M0s))


def kernel(class_logits, box_regression, proposals):
    f32 = jnp.float32
    lg = class_logits.astype(f32).reshape(_B, _N, _C).transpose(0, 2, 1)
    lg = jnp.pad(lg, ((0, 0), (0, _CP - _C), (0, _NP - _N)),
                 constant_values=-1e9).reshape(_B * _CP * 8, 128)
    d = box_regression.astype(f32).reshape(_B, _N, _C, 4).transpose(3, 0, 2, 1)
    d = jnp.pad(d, ((0, 0), (0, 0), (0, _CP - _C), (0, _NP - _N)))
    d = d.reshape(4 * _B * _CP * 8, 128)
    pr = proposals.astype(f32).transpose(0, 2, 1)
    pr = jnp.pad(pr, ((0, 0), (0, 0), (0, _NP - _N))).reshape(_B * 4 * 8, 128)

    out = pl.pallas_call(
        _fused,
        out_shape=jax.ShapeDtypeStruct((_B, _CP, _CP), f32),
        scratch_shapes=[pltpu.VMEM((_CP * 8, 128), f32)] * (6 * _B),
    )(lg, d, pr)

    res = out[:, :_DETS, :]
    sel_boxes = res[..., 0:4]
    sel_scores = res[..., 4]
    sel_labels = res[..., 5].astype(jnp.int32)
    keep = res[..., 6].astype(jnp.int32)
    valid = res[..., 7] > 0.5
    return sel_boxes, sel_scores, sel_labels, keep, valid


# speedup vs baseline: 71.0654x; 1.6137x over previous
import jax
import jax.numpy as jnp
import math
from jax.experimental import pallas as pl
from jax.experimental.pallas import tpu as pltpu

_B = 2
_N = 1000
_NP = 1024
_C = 91
_CP = 128
_IMG = 800.0
_SCORE_THRESH = 0.05
_NMS_THRESH = 0.5
_DETS = 100
_MIN_SIZE = 0.01
_CLIP = math.log(1000.0 / 16.0)
_BIG = 1 << 30


def _fused(lg_ref, d_ref, props_ref, out_ref, *scratch):
    f32 = jnp.float32
    refs = [scratch[b * 6:(b + 1) * 6] for b in range(_B)]
    l = lg_ref[...].reshape(_B, _CP, 8, 128)
    m = jnp.max(l, axis=1, keepdims=True)
    e = jnp.exp(l - m)
    p = e / jnp.sum(e, axis=1, keepdims=True)

    pr = props_ref[...].reshape(_B, 4, 1, 8, 128)
    p0 = pr[:, 0]
    p1 = pr[:, 1]
    p2 = pr[:, 2]
    p3 = pr[:, 3]
    w = (p2 - p0)
    h = (p3 - p1)
    cx = p0 + 0.5 * w
    cy = p1 + 0.5 * h
    d = d_ref[...].reshape(4, _B, _CP, 8, 128)
    dx = d[0] / 10.0
    dy = d[1] / 10.0
    dw = jnp.minimum(d[2] / 5.0, _CLIP)
    dh = jnp.minimum(d[3] / 5.0, _CLIP)
    pcx = dx * w + cx
    pcy = dy * h + cy
    pw = jnp.exp(dw) * w
    ph = jnp.exp(dh) * h
    x1 = jnp.clip(pcx - 0.5 * pw, 0.0, _IMG)
    y1 = jnp.clip(pcy - 0.5 * ph, 0.0, _IMG)
    x2 = jnp.clip(pcx + 0.5 * pw, 0.0, _IMG)
    y2 = jnp.clip(pcy + 0.5 * ph, 0.0, _IMG)

    crow = jax.lax.broadcasted_iota(jnp.int32, (_CP, 8, 128), 0)
    nidx = jax.lax.broadcasted_iota(jnp.int32, (_CP, 8, 128), 1) * 128 + \
        jax.lax.broadcasted_iota(jnp.int32, (_CP, 8, 128), 2)
    ok_pos = (crow >= 1) & (crow <= _C - 1) & (nidx < _N)
    offs = crow.astype(f32) * (_IMG + 2.0)

    lane = jax.lax.broadcasted_iota(jnp.int32, (1, _CP), 1)
    tidx = jax.lax.broadcasted_iota(jnp.int32, (8, 128), 0) * 128 + \
        jax.lax.broadcasted_iota(jnp.int32, (8, 128), 1)
    ones = jnp.ones((128, 128), f32)

    carries = []
    for b in range(_B):
        s_ref, xo1_ref, yo1_ref, xo2_ref, yo2_ref, ar_ref = refs[b]
        mask = (p[b] > _SCORE_THRESH) & ((x2[b] - x1[b]) >= _MIN_SIZE) & \
               ((y2[b] - y1[b]) >= _MIN_SIZE) & ok_pos
        s = jnp.where(mask, p[b], -1.0)
        s_ref[...] = s.reshape(_CP * 8, 128)
        xo1 = x1[b] + offs
        yo1 = y1[b] + offs
        xo2 = x2[b] + offs
        yo2 = y2[b] + offs
        xo1_ref[...] = xo1.reshape(_CP * 8, 128)
        yo1_ref[...] = yo1.reshape(_CP * 8, 128)
        xo2_ref[...] = xo2.reshape(_CP * 8, 128)
        yo2_ref[...] = yo2.reshape(_CP * 8, 128)
        ar_ref[...] = ((xo2 - xo1) * (yo2 - yo1)).reshape(_CP * 8, 128)
        M = jnp.max(s, axis=(1, 2)).reshape(1, _CP)
        mx0 = jnp.max(M)
        cc0 = jnp.min(jnp.where(M == mx0, lane, _BIG))
        carries.append((M, cc0, mx0))

    def one_image(b, carry, t):
        s_ref, xo1_ref, yo1_ref, xo2_ref, yo2_ref, ar_ref = refs[b]
        M, ccv, mxv = carry
        Mrest = jnp.where(lane == ccv, -2.0, M)
        mx_rest = jnp.max(Mrest)
        cc_rest = jnp.min(jnp.where(Mrest == mx_rest, lane, _BIG))

        base = ccv * 8
        srow = s_ref[pl.ds(base, 8), :]
        eq = srow == mxv
        ok = mxv > 0.0

        ro1 = xo1_ref[pl.ds(base, 8), :]
        ro2 = yo1_ref[pl.ds(base, 8), :]
        ro3 = xo2_ref[pl.ds(base, 8), :]
        ro4 = yo2_ref[pl.ds(base, 8), :]
        cat = jnp.concatenate(
            [jnp.where(eq, ro1, 0.0), jnp.where(eq, ro2, 0.0),
             jnp.where(eq, ro3, 0.0), jnp.where(eq, ro4, 0.0)], axis=0)
        sums = jax.lax.dot_general(cat, ones, (((1,), (0,)), ((), ())),
                                   preferred_element_type=f32)
        bs = jnp.sum(sums.reshape(4, 8, 128), axis=1)
        bx1 = jnp.broadcast_to(bs[0:1, :], (8, 128))
        by1 = jnp.broadcast_to(bs[1:2, :], (8, 128))
        bx2 = jnp.broadcast_to(bs[2:3, :], (8, 128))
        by2 = jnp.broadcast_to(bs[3:4, :], (8, 128))
        ref_area = (bx2 - bx1) * (by2 - by1)

        xx1 = jnp.maximum(bx1, ro1)
        yy1 = jnp.maximum(by1, ro2)
        xx2 = jnp.minimum(bx2, ro3)
        yy2 = jnp.minimum(by2, ro4)
        inter = jnp.maximum(xx2 - xx1, 0.0) * jnp.maximum(yy2 - yy1, 0.0)
        union = ar_ref[pl.ds(base, 8), :] + ref_area - inter
        iou = jnp.where(union > 0.0, inter / jnp.maximum(union, 1e-9), 0.0)
        supp = (iou > _NMS_THRESH) & ok
        srow_new = jnp.where(supp, -1.0, srow)
        s_ref[pl.ds(base, 8), :] = srow_new
        rowmax = jnp.max(srow_new)

        M_new = jnp.where(lane == ccv, rowmax, M)
        mx_next = jnp.maximum(mx_rest, rowmax)
        cc_next = jnp.where(rowmax >= mx_rest, ccv, cc_rest)

        nn = jnp.min(jnp.where(eq, tidx, _BIG))
        off_c = ccv.astype(f32) * (_IMG + 2.0)
        vf = jnp.where(ok, 1.0, 0.0).astype(f32)
        vals = [(bs[0, 0] - off_c) * vf, (bs[1, 0] - off_c) * vf,
                (bs[2, 0] - off_c) * vf, (bs[3, 0] - off_c) * vf,
                jnp.where(ok, mxv, 0.0),
                jnp.where(ok, ccv, 0).astype(f32),
                jnp.where(ok, nn * 90 + ccv - 1, 0).astype(f32),
                vf]
        for j, v in enumerate(vals):
            out_ref[b, pl.ds(t, 1), j:j + 1] = v.reshape(1, 1)
        return (M_new, cc_next, mx_next)

    def body(t, cs):
        return tuple(one_image(b, cs[b], t) for b in range(_B))

    jax.lax.fori_loop(0, _DETS, body, tuple(carries))


def kernel(class_logits, box_regression, proposals):
    f32 = jnp.float32
    lg = class_logits.astype(f32).reshape(_B, _N, _C).transpose(0, 2, 1)
    lg = jnp.pad(lg, ((0, 0), (0, _CP - _C), (0, _NP - _N)),
                 constant_values=-1e9).reshape(_B * _CP * 8, 128)
    d = box_regression.astype(f32).reshape(_B, _N, _C, 4).transpose(3, 0, 2, 1)
    d = jnp.pad(d, ((0, 0), (0, 0), (0, _CP - _C), (0, _NP - _N)))
    d = d.reshape(4 * _B * _CP * 8, 128)
    pr = proposals.astype(f32).transpose(0, 2, 1)
    pr = jnp.pad(pr, ((0, 0), (0, 0), (0, _NP - _N))).reshape(_B * 4 * 8, 128)

    out = pl.pallas_call(
        _fused,
        out_shape=jax.ShapeDtypeStruct((_B, _CP, _CP), f32),
        scratch_shapes=[pltpu.VMEM((_CP * 8, 128), f32)] * (6 * _B),
    )(lg, d, pr)

    res = out[:, :_DETS, :]
    sel_boxes = res[..., 0:4]
    sel_scores = res[..., 4]
    sel_labels = res[..., 5].astype(jnp.int32)
    keep = res[..., 6].astype(jnp.int32)
    valid = res[..., 7] > 0.5
    return sel_boxes, sel_scores, sel_labels, keep, valid
